# Initial kernel scaffold; baseline (speedup 1.0000x reference)
import jax
import jax.numpy as jnp
from jax.experimental import pallas as pl


def kernel(idxes, pe):
    raise NotImplementedError("write your pallas kernel here")


# speedup vs baseline: 2.1862x; 2.1862x over previous
import functools

import jax
import jax.numpy as jnp
from jax import lax
from jax.experimental import pallas as pl
from jax.experimental.pallas import tpu as pltpu
from jax.experimental.pallas import tpu_sc as plsc

D = 1024
NC = 2
NS = 16
NW = NC * NS
C = 64


def _sc_gather(idx_flat, pe):
    b_total = idx_flat.shape[0]
    bpw = b_total // NW
    nchunk = bpw // C

    mesh = plsc.VectorSubcoreMesh(core_axis_name="c", subcore_axis_name="s")

    @functools.partial(
        pl.kernel,
        mesh=mesh,
        out_type=jax.ShapeDtypeStruct((b_total, D), jnp.float32),
        scratch_types=[
            pltpu.VMEM((bpw,), jnp.int32),
            pltpu.VMEM((C, D), jnp.float32),
            pltpu.SemaphoreType.DMA,
        ],
    )
    def k(idx_hbm, table_hbm, out_hbm, idx_v, rows_v, sem):
        wid = lax.axis_index("s") * NC + lax.axis_index("c")
        base = wid * bpw
        pltpu.sync_copy(idx_hbm.at[pl.ds(base, bpw)], idx_v)

        def body(c, carry):
            pltpu.async_copy(
                table_hbm.at[idx_v.at[pl.ds(c * C, C)]], rows_v, sem
            ).wait()
            pltpu.sync_copy(rows_v, out_hbm.at[pl.ds(base + c * C, C)])
            return carry

        lax.fori_loop(0, nchunk, body, 0)

    return k(idx_flat, pe)


def kernel(idxes, pe):
    out = _sc_gather(idxes.reshape(-1).astype(jnp.int32), pe)
    return out.reshape(idxes.shape + (D,))


# speedup vs baseline: 2.3855x; 1.0911x over previous
import functools

import jax
import jax.numpy as jnp
from jax import lax
from jax.experimental import pallas as pl
from jax.experimental.pallas import tpu as pltpu
from jax.experimental.pallas import tpu_sc as plsc

D = 1024
NC = 2
NS = 16
NW = NC * NS
C = 32


def _sc_gather(idx_flat, pe):
    b_total = idx_flat.shape[0]
    bpw = b_total // NW
    nchunk = bpw // C
    npair = nchunk // 2

    mesh = plsc.VectorSubcoreMesh(core_axis_name="c", subcore_axis_name="s")

    @functools.partial(
        pl.kernel,
        mesh=mesh,
        out_type=jax.ShapeDtypeStruct((b_total, D), jnp.float32),
        scratch_types=[
            pltpu.VMEM((bpw,), jnp.int32),
            pltpu.VMEM((C, D), jnp.float32),
            pltpu.VMEM((C, D), jnp.float32),
            pltpu.SemaphoreType.DMA,
            pltpu.SemaphoreType.DMA,
        ],
    )
    def k(idx_hbm, table_hbm, out_hbm, idx_v, buf0, buf1, sem0, sem1):
        wid = lax.axis_index("s") * NC + lax.axis_index("c")
        base = wid * bpw
        pltpu.sync_copy(idx_hbm.at[pl.ds(base, bpw)], idx_v)

        def gather(c, buf, sem):
            return pltpu.async_copy(
                table_hbm.at[idx_v.at[pl.ds(c * C, C)]], buf, sem
            )

        gather(0, buf0, sem0)

        def body(i, carry):
            c0 = 2 * i
            gather(c0 + 1, buf1, sem1)
            pltpu.make_async_copy(table_hbm.at[pl.ds(0, C)], buf0, sem0).wait()
            pltpu.sync_copy(buf0, out_hbm.at[pl.ds(base + c0 * C, C)])

            @pl.when(c0 + 2 < nchunk)
            def _():
                gather(c0 + 2, buf0, sem0)

            pltpu.make_async_copy(table_hbm.at[pl.ds(0, C)], buf1, sem1).wait()
            pltpu.sync_copy(buf1, out_hbm.at[pl.ds(base + (c0 + 1) * C, C)])
            return carry

        lax.fori_loop(0, npair, body, 0)

    return k(idx_flat, pe)


def kernel(idxes, pe):
    out = _sc_gather(idxes.reshape(-1).astype(jnp.int32), pe)
    return out.reshape(idxes.shape + (D,))


# speedup vs baseline: 2.3886x; 1.0013x over previous
import functools

import jax
import jax.numpy as jnp
from jax import lax
from jax.experimental import pallas as pl
from jax.experimental.pallas import tpu as pltpu
from jax.experimental.pallas import tpu_sc as plsc

D = 1024
NC = 2
NS = 16
NW = NC * NS
C = 16
NBUF = 4
LA = 2


def _sc_gather(idx_flat, pe):
    b_total = idx_flat.shape[0]
    bpw = b_total // NW
    nchunk = bpw // C
    niter = nchunk // NBUF

    mesh = plsc.VectorSubcoreMesh(core_axis_name="c", subcore_axis_name="s")

    @functools.partial(
        pl.kernel,
        mesh=mesh,
        out_type=jax.ShapeDtypeStruct((b_total, D), jnp.float32),
        scratch_types=[
            pltpu.VMEM((bpw,), jnp.int32),
        ]
        + [pltpu.VMEM((C, D), jnp.float32) for _ in range(NBUF)]
        + [pltpu.SemaphoreType.DMA for _ in range(2 * NBUF)],
    )
    def k(idx_hbm, table_hbm, out_hbm, idx_v, *rest):
        bufs = rest[:NBUF]
        gsems = rest[NBUF:2 * NBUF]
        psems = rest[2 * NBUF:]
        wid = lax.axis_index("s") * NC + lax.axis_index("c")
        base = wid * bpw
        pltpu.sync_copy(idx_hbm.at[pl.ds(base, bpw)], idx_v)

        def gather(w, j):
            pltpu.async_copy(
                table_hbm.at[idx_v.at[pl.ds(w * C, C)]], bufs[j], gsems[j]
            )

        def put(v, j):
            pltpu.async_copy(bufs[j], out_hbm.at[pl.ds(base + v * C, C)],
                             psems[j])

        def wait_g(j):
            pltpu.make_async_copy(table_hbm.at[pl.ds(0, C)], bufs[j],
                                  gsems[j]).wait()

        def wait_p(j):
            pltpu.make_async_copy(bufs[j], out_hbm.at[pl.ds(0, C)],
                                  psems[j]).wait()

        for w in range(LA):
            gather(w, w)

        for b in range(NBUF):
            wait_g(b)
            put(b, b)
            w = b + LA
            if w >= NBUF:
                wait_p(w % NBUF)
            gather(w, w % NBUF)

        def body(i, carry):
            for b in range(NBUF):
                v = NBUF * i + b
                wait_g(b)
                put(v, b)
                bw = (b + LA) % NBUF
                wait_p(bw)
                gather(v + LA, bw)
            return carry

        lax.fori_loop(1, niter - 1, body, 0)

        for b in range(NBUF):
            v = nchunk - NBUF + b
            wait_g(b)
            put(v, b)
            if b + LA < NBUF:
                bw = (b + LA) % NBUF
                wait_p(bw)
                gather(v + LA, bw)

        for b in range(NBUF):
            wait_p(b)

    return k(idx_flat, pe)


def kernel(idxes, pe):
    out = _sc_gather(idxes.reshape(-1).astype(jnp.int32), pe)
    return out.reshape(idxes.shape + (D,))
